# baseline (speedup 1.0000x reference)
import functools

import jax
import jax.numpy as jnp
from jax import lax
from jax.experimental import pallas as pl
from jax.experimental.pallas import tpu as pltpu
from jax.experimental.pallas import tpu_sc as plsc

VOCAB = 1000000
HALF = 1 << 19
B = 4096
S = 200
D = 64
NUM_CLASSES = 1000
L = 16
NC = 2
NS = 16
NW = NC * NS
NB = B // NW
CHUNK = 2
NCHUNK = NB // CHUNK
IDXW = 80
NSTREAM = CHUNK * S // IDXW
TOK = CHUNK * S


def _format_table(tableT):
    BN = 4096
    ngrid = HALF // BN

    def tr(a_ref, b_ref, o_ref):
        o_ref[:, 0:D] = a_ref[...].T.astype(jnp.bfloat16)
        o_ref[:, D:2 * D] = b_ref[...].T.astype(jnp.bfloat16)

    last = pl.cdiv(VOCAB, BN) - 1

    return pl.pallas_call(
        tr,
        grid=(ngrid,),
        in_specs=[
            pl.BlockSpec((D, BN), lambda i: (0, i)),
            pl.BlockSpec((D, BN), lambda i: (0, jnp.minimum(i + ngrid, last))),
        ],
        out_specs=pl.BlockSpec((BN, 2 * D), lambda i: (i, 0)),
        out_shape=jax.ShapeDtypeStruct((HALF, 2 * D), jnp.bfloat16),
    )(tableT, tableT)


def _sc_pool(table2, idx1d, cidx0, cidx1, cidx2, cat0, cat1, cat2):
    mesh = plsc.VectorSubcoreMesh(
        core_axis_name="c", subcore_axis_name="s",
        num_cores=NC, num_subcores=NS)

    @functools.partial(
        pl.kernel,
        out_type=jax.ShapeDtypeStruct((B, D), jnp.float32),
        mesh=mesh,
        compiler_params=pltpu.CompilerParams(
            needs_layout_passes=False, use_tc_tiling_on_sc=False),
        scratch_types=[
            pltpu.VMEM((TOK,), jnp.int32),
            pltpu.VMEM((TOK, D), jnp.bfloat16),
            pltpu.VMEM((NB,), jnp.int32),
            pltpu.VMEM((NB, D), jnp.float32),
            pltpu.VMEM((NB, D), jnp.float32),
            pltpu.VMEM((NB, D), jnp.float32),
            pltpu.VMEM((NB, D), jnp.float32),
            pltpu.SemaphoreType.DMA,
        ],
    )
    def k(table_h, idx_h, c0i_h, c1i_h, c2i_h, cat0_h, cat1_h, cat2_h,
          out_h, idxv, rows, cidxv, cr0, cr1, cr2, pooled, sem):
        wid = lax.axis_index("s") * NC + lax.axis_index("c")
        lane = lax.iota(jnp.int32, L)

        def seg_count(r):
            cv = jnp.zeros((L,), jnp.int32)
            for t in range(S // L):
                v = idxv[pl.ds(r * S + L * t, L)]
                cv = cv + plsc.all_reduce_population_count(v != 0)
            tail = S % L
            v = idxv[pl.ds(r * S + S - L, L)]
            cv = cv + plsc.all_reduce_population_count((lane >= L - tail) & (v != 0))
            return cv

        def row_accum(r):
            def tbody(t, a):
                a0, a1, a2, a3 = a
                base = r * S + t * 8
                for u in range(8):
                    tt = base + u
                    lo = rows[tt, pl.ds(0, 2 * L)]
                    hi = rows[tt, pl.ds(2 * L, 2 * L)]
                    e0, o0 = plsc.unpack(lo, format=plsc.PackFormat.INTERLEAVED)
                    e1, o1 = plsc.unpack(hi, format=plsc.PackFormat.INTERLEAVED)
                    a0 = a0 + e0
                    a1 = a1 + o0
                    a2 = a2 + e1
                    a3 = a3 + o1
                return (a0, a1, a2, a3)
            z = jnp.zeros((L,), jnp.float32)
            return lax.fori_loop(0, S // 8, tbody, (z, z, z, z))

        def chunk(i, carry):
            ib = wid * (NB * S) + i * TOK
            pltpu.sync_copy(idx_h.at[pl.ds(ib, TOK)], idxv)
            for t in range(TOK // L):
                v = idxv[pl.ds(L * t, L)]
                idxv[pl.ds(L * t, L)] = ((v << 1) | (v >> 19)) & (2 * HALF - 1)
            cps = [
                pltpu.async_copy(
                    table_h.at[idxv.at[pl.ds(j * IDXW, IDXW)]],
                    rows.at[pl.ds(j * IDXW, IDXW)], sem)
                for j in range(NSTREAM)
            ]
            for cp in cps:
                cp.wait()
            for r in range(CHUNK):
                a0, a1, a2, a3 = row_accum(r)
                cv = seg_count(r)
                inv = jnp.where(cv > 0, 1.0 / cv.astype(jnp.float32), 0.0)
                row = i * CHUNK + r
                pooled[row, pl.ds(0, L)] = a0 * inv
                pooled[row, pl.ds(L, L)] = a1 * inv
                pooled[row, pl.ds(2 * L, L)] = a2 * inv
                pooled[row, pl.ds(3 * L, L)] = a3 * inv
            return carry

        lax.fori_loop(0, NCHUNK, chunk, 0)

        base = wid * NB
        pltpu.sync_copy(c0i_h.at[pl.ds(base, NB)], cidxv)
        pltpu.async_copy(cat0_h.at[cidxv], cr0, sem).wait()
        pltpu.sync_copy(c1i_h.at[pl.ds(base, NB)], cidxv)
        pltpu.async_copy(cat1_h.at[cidxv], cr1, sem).wait()
        pltpu.sync_copy(c2i_h.at[pl.ds(base, NB)], cidxv)
        pltpu.async_copy(cat2_h.at[cidxv], cr2, sem).wait()

        def cbody(r, carry):
            for j in range(D // L):
                sl = pl.ds(L * j, L)
                pooled[r, sl] = pooled[r, sl] + cr0[r, sl] + cr1[r, sl] + cr2[r, sl]
            return carry

        lax.fori_loop(0, NB, cbody, 0)
        pltpu.sync_copy(pooled, out_h.at[pl.ds(base, NB)])

    return k(table2, idx1d, cidx0, cidx1, cidx2, cat0, cat1, cat2)


def _linear(x, W, b):
    BM = 512

    def mm(x_ref, w_ref, b_ref, o_ref):
        o_ref[...] = lax.dot_general(
            x_ref[...], w_ref[...], (((1,), (1,)), ((), ())),
            preferred_element_type=jnp.float32) + b_ref[...]

    return pl.pallas_call(
        mm,
        grid=(B // BM,),
        in_specs=[
            pl.BlockSpec((BM, D), lambda i: (i, 0)),
            pl.BlockSpec((NUM_CLASSES, D), lambda i: (0, 0)),
            pl.BlockSpec((1, NUM_CLASSES), lambda i: (0, 0)),
        ],
        out_specs=pl.BlockSpec((BM, NUM_CLASSES), lambda i: (i, 0)),
        out_shape=jax.ShapeDtypeStruct((B, NUM_CLASSES), jnp.float32),
    )(x, W, b.reshape(1, NUM_CLASSES))


_PERM = tuple(range(0, 2 * L, 2)) + tuple(range(1, 2 * L, 2)) + \
    tuple(range(2 * L, 4 * L, 2)) + tuple(range(2 * L + 1, 4 * L, 2))


def kernel(encoded_text, additional_inputs, emb_table, cat0, cat1, cat2, W, b):
    t128 = _format_table(emb_table.T)
    table2 = t128.reshape(2 * HALF, D)
    idx1d = encoded_text.reshape(B * S)
    cidx0 = additional_inputs[:, 0]
    cidx1 = additional_inputs[:, 1]
    cidx2 = additional_inputs[:, 2]
    perm = jnp.array(_PERM, dtype=jnp.int32)
    pooled = _sc_pool(table2, idx1d, cidx0, cidx1, cidx2,
                      cat0[:, perm], cat1[:, perm], cat2[:, perm])
    return _linear(pooled, W[:, perm], b)


# speedup vs baseline: 1.7664x; 1.7664x over previous
import functools

import jax
import jax.numpy as jnp
from jax import lax
from jax.experimental import pallas as pl
from jax.experimental.pallas import tpu as pltpu
from jax.experimental.pallas import tpu_sc as plsc

VOCAB = 1000000
QUART = 1 << 18
B = 4096
S = 200
D = 64
NUM_CLASSES = 1000
L = 16
NC = 2
NS = 16
NW = NC * NS
NB = B // NW
CHUNK = 2
NCHUNK = NB // CHUNK
IDXW = 80
NSTREAM = CHUNK * S // IDXW
TOK = CHUNK * S


def _format_table(tableT):
    BN = 4096
    ngrid = QUART // BN
    last = pl.cdiv(VOCAB, BN) - 1

    def tr(a0_ref, a1_ref, a2_ref, a3_ref, o_ref):
        for q, a_ref in enumerate((a0_ref, a1_ref, a2_ref, a3_ref)):
            w = pltpu.bitcast(a_ref[...].astype(jnp.bfloat16), jnp.int32)
            o_ref[:, (D // 2) * q:(D // 2) * (q + 1)] = w.T

    def mk_map(q):
        return lambda i: (0, jnp.minimum(i + q * ngrid, last))

    return pl.pallas_call(
        tr,
        grid=(ngrid,),
        in_specs=[pl.BlockSpec((D, BN), mk_map(q)) for q in range(4)],
        out_specs=pl.BlockSpec((BN, 2 * D), lambda i: (i, 0)),
        out_shape=jax.ShapeDtypeStruct((QUART, 2 * D), jnp.int32),
    )(tableT, tableT, tableT, tableT)


def _sc_pool(table2, idx1d, cidx0, cidx1, cidx2, cat0, cat1, cat2):
    mesh = plsc.VectorSubcoreMesh(
        core_axis_name="c", subcore_axis_name="s",
        num_cores=NC, num_subcores=NS)

    @functools.partial(
        pl.kernel,
        out_type=jax.ShapeDtypeStruct((B, D), jnp.float32),
        mesh=mesh,
        compiler_params=pltpu.CompilerParams(
            needs_layout_passes=False, use_tc_tiling_on_sc=False),
        scratch_types=[
            pltpu.VMEM((TOK,), jnp.int32),
            pltpu.VMEM((TOK, D // 2), jnp.int32),
            pltpu.VMEM((NB,), jnp.int32),
            pltpu.VMEM((NB, D), jnp.float32),
            pltpu.VMEM((NB, D), jnp.float32),
            pltpu.VMEM((NB, D), jnp.float32),
            pltpu.VMEM((NB, D), jnp.float32),
            pltpu.SemaphoreType.DMA,
        ],
    )
    def k(table_h, idx_h, c0i_h, c1i_h, c2i_h, cat0_h, cat1_h, cat2_h,
          out_h, idxv, rows, cidxv, cr0, cr1, cr2, pooled, sem):
        wid = lax.axis_index("s") * NC + lax.axis_index("c")
        lane = lax.iota(jnp.int32, L)

        def seg_count(r):
            cv = jnp.zeros((L,), jnp.int32)
            for t in range(S // L):
                v = idxv[pl.ds(r * S + L * t, L)]
                cv = cv + plsc.all_reduce_population_count(v != 0)
            tail = S % L
            v = idxv[pl.ds(r * S + S - L, L)]
            cv = cv + plsc.all_reduce_population_count((lane >= L - tail) & (v != 0))
            return cv

        def row_accum(r):
            def tbody(t, a):
                a0, a1, a2, a3 = a
                base = r * S + t * 8
                for u in range(8):
                    tt = base + u
                    lo = plsc.bitcast(rows[tt, pl.ds(0, L)], jnp.bfloat16)
                    hi = plsc.bitcast(rows[tt, pl.ds(L, L)], jnp.bfloat16)
                    e0, o0 = plsc.unpack(lo, format=plsc.PackFormat.INTERLEAVED)
                    e1, o1 = plsc.unpack(hi, format=plsc.PackFormat.INTERLEAVED)
                    a0 = a0 + e0
                    a1 = a1 + o0
                    a2 = a2 + e1
                    a3 = a3 + o1
                return (a0, a1, a2, a3)
            z = jnp.zeros((L,), jnp.float32)
            return lax.fori_loop(0, S // 8, tbody, (z, z, z, z))

        def chunk(i, carry):
            ib = wid * (NB * S) + i * TOK
            pltpu.sync_copy(idx_h.at[pl.ds(ib, TOK)], idxv)
            for t in range(TOK // L):
                v = idxv[pl.ds(L * t, L)]
                idxv[pl.ds(L * t, L)] = \
                    ((v << 2) | (v >> 18)) & (4 * QUART - 1)
            cps = [
                pltpu.async_copy(
                    table_h.at[idxv.at[pl.ds(j * IDXW, IDXW)]],
                    rows.at[pl.ds(j * IDXW, IDXW)], sem)
                for j in range(NSTREAM)
            ]
            for cp in cps:
                cp.wait()
            for r in range(CHUNK):
                a0, a1, a2, a3 = row_accum(r)
                cv = seg_count(r)
                inv = jnp.where(cv > 0, 1.0 / cv.astype(jnp.float32), 0.0)
                row = i * CHUNK + r
                pooled[row, pl.ds(0, L)] = a0 * inv
                pooled[row, pl.ds(L, L)] = a1 * inv
                pooled[row, pl.ds(2 * L, L)] = a2 * inv
                pooled[row, pl.ds(3 * L, L)] = a3 * inv
            return carry

        lax.fori_loop(0, NCHUNK, chunk, 0)

        base = wid * NB
        pltpu.sync_copy(c0i_h.at[pl.ds(base, NB)], cidxv)
        pltpu.async_copy(cat0_h.at[cidxv], cr0, sem).wait()
        pltpu.sync_copy(c1i_h.at[pl.ds(base, NB)], cidxv)
        pltpu.async_copy(cat1_h.at[cidxv], cr1, sem).wait()
        pltpu.sync_copy(c2i_h.at[pl.ds(base, NB)], cidxv)
        pltpu.async_copy(cat2_h.at[cidxv], cr2, sem).wait()

        def cbody(r, carry):
            for j in range(D // L):
                sl = pl.ds(L * j, L)
                pooled[r, sl] = pooled[r, sl] + cr0[r, sl] + cr1[r, sl] + cr2[r, sl]
            return carry

        lax.fori_loop(0, NB, cbody, 0)
        pltpu.sync_copy(pooled, out_h.at[pl.ds(base, NB)])

    return k(table2, idx1d, cidx0, cidx1, cidx2, cat0, cat1, cat2)


def _linear(x, W, b):
    BM = 512

    def mm(x_ref, w_ref, b_ref, o_ref):
        o_ref[...] = lax.dot_general(
            x_ref[...], w_ref[...], (((1,), (1,)), ((), ())),
            preferred_element_type=jnp.float32) + b_ref[...]

    return pl.pallas_call(
        mm,
        grid=(B // BM,),
        in_specs=[
            pl.BlockSpec((BM, D), lambda i: (i, 0)),
            pl.BlockSpec((NUM_CLASSES, D), lambda i: (0, 0)),
            pl.BlockSpec((1, NUM_CLASSES), lambda i: (0, 0)),
        ],
        out_specs=pl.BlockSpec((BM, NUM_CLASSES), lambda i: (i, 0)),
        out_shape=jax.ShapeDtypeStruct((B, NUM_CLASSES), jnp.float32),
    )(x, W, b.reshape(1, NUM_CLASSES))


_PERM = tuple(range(0, 2 * L, 2)) + tuple(range(1, 2 * L, 2)) + \
    tuple(range(2 * L, 4 * L, 2)) + tuple(range(2 * L + 1, 4 * L, 2))


def kernel(encoded_text, additional_inputs, emb_table, cat0, cat1, cat2, W, b):
    t128 = _format_table(emb_table.T)
    table2 = t128.reshape(4 * QUART, D // 2)
    idx1d = encoded_text.reshape(B * S)
    cidx0 = additional_inputs[:, 0]
    cidx1 = additional_inputs[:, 1]
    cidx2 = additional_inputs[:, 2]
    perm = jnp.array(_PERM, dtype=jnp.int32)
    pooled = _sc_pool(table2, idx1d, cidx0, cidx1, cidx2,
                      cat0[:, perm], cat1[:, perm], cat2[:, perm])
    return _linear(pooled, W[:, perm], b)


# speedup vs baseline: 2.0462x; 1.1584x over previous
import functools

import jax
import jax.numpy as jnp
from jax import lax
from jax.experimental import pallas as pl
from jax.experimental.pallas import tpu as pltpu
from jax.experimental.pallas import tpu_sc as plsc

VOCAB = 1000000
QUART = 1 << 18
B = 4096
S = 200
D = 64
NUM_CLASSES = 1000
L = 16
NC = 2
NS = 16
NW = NC * NS
NB = B // NW
CHUNK = 2
NCHUNK = NB // CHUNK
IDXW = 80
NSTREAM = CHUNK * S // IDXW
TOK = CHUNK * S


def _format_table(tableT):
    BN = 4096
    ngrid = QUART // BN
    last = pl.cdiv(VOCAB, BN) - 1

    def tr(a0_ref, a1_ref, a2_ref, a3_ref, o_ref):
        for q, a_ref in enumerate((a0_ref, a1_ref, a2_ref, a3_ref)):
            w = pltpu.bitcast(a_ref[...].astype(jnp.bfloat16), jnp.int32)
            o_ref[:, (D // 2) * q:(D // 2) * (q + 1)] = w.T

    def mk_map(q):
        return lambda i: (0, jnp.minimum(i + q * ngrid, last))

    return pl.pallas_call(
        tr,
        grid=(ngrid,),
        in_specs=[pl.BlockSpec((D, BN), mk_map(q)) for q in range(4)],
        out_specs=pl.BlockSpec((BN, 2 * D), lambda i: (i, 0)),
        out_shape=jax.ShapeDtypeStruct((QUART, 2 * D), jnp.int32),
    )(tableT, tableT, tableT, tableT)


def _sc_pool(table2, idx1d, cidx0, cidx1, cidx2, cat0, cat1, cat2):
    mesh = plsc.VectorSubcoreMesh(
        core_axis_name="c", subcore_axis_name="s",
        num_cores=NC, num_subcores=NS)

    @functools.partial(
        pl.kernel,
        out_type=jax.ShapeDtypeStruct((B, D), jnp.float32),
        mesh=mesh,
        compiler_params=pltpu.CompilerParams(
            needs_layout_passes=False, use_tc_tiling_on_sc=False),
        scratch_types=[
            pltpu.VMEM((TOK,), jnp.int32),
            pltpu.VMEM((TOK,), jnp.int32),
            pltpu.VMEM((TOK, D // 2), jnp.int32),
            pltpu.VMEM((TOK, D // 2), jnp.int32),
            pltpu.VMEM((NB,), jnp.int32),
            pltpu.VMEM((NB, D), jnp.float32),
            pltpu.VMEM((NB, D), jnp.float32),
            pltpu.VMEM((NB, D), jnp.float32),
            pltpu.VMEM((NB, D), jnp.float32),
            pltpu.SemaphoreType.DMA,
            pltpu.SemaphoreType.DMA,
        ],
    )
    def k(table_h, idx_h, c0i_h, c1i_h, c2i_h, cat0_h, cat1_h, cat2_h,
          out_h, idxv0, idxv1, rows0, rows1, cidxv, cr0, cr1, cr2, pooled,
          sem0, sem1):
        wid = lax.axis_index("s") * NC + lax.axis_index("c")
        lane = lax.iota(jnp.int32, L)

        def seg_count(idxv, r):
            cv = jnp.zeros((L,), jnp.int32)
            for t in range(S // L):
                v = idxv[pl.ds(r * S + L * t, L)]
                cv = cv + plsc.all_reduce_population_count(v != 0)
            tail = S % L
            v = idxv[pl.ds(r * S + S - L, L)]
            cv = cv + plsc.all_reduce_population_count((lane >= L - tail) & (v != 0))
            return cv

        def row_accum(rows, r):
            def tbody(t, a):
                a0, a1, a2, a3 = a
                base = r * S + t * 8
                for u in range(8):
                    tt = base + u
                    lo = plsc.bitcast(rows[tt, pl.ds(0, L)], jnp.bfloat16)
                    hi = plsc.bitcast(rows[tt, pl.ds(L, L)], jnp.bfloat16)
                    e0, o0 = plsc.unpack(lo, format=plsc.PackFormat.INTERLEAVED)
                    e1, o1 = plsc.unpack(hi, format=plsc.PackFormat.INTERLEAVED)
                    a0 = a0 + e0
                    a1 = a1 + o0
                    a2 = a2 + e1
                    a3 = a3 + o1
                return (a0, a1, a2, a3)
            z = jnp.zeros((L,), jnp.float32)
            return lax.fori_loop(0, S // 8, tbody, (z, z, z, z))

        def stage(g, idxv, rows, sem):
            ib = wid * (NB * S) + g * TOK
            pltpu.sync_copy(idx_h.at[pl.ds(ib, TOK)], idxv)
            for t in range(TOK // L):
                v = idxv[pl.ds(L * t, L)]
                idxv[pl.ds(L * t, L)] = \
                    ((v << 2) | (v >> 18)) & (4 * QUART - 1)
            for j in range(NSTREAM):
                pltpu.async_copy(
                    table_h.at[idxv.at[pl.ds(j * IDXW, IDXW)]],
                    rows.at[pl.ds(j * IDXW, IDXW)], sem)

        def drain(rows, sem):
            for j in range(NSTREAM):
                pltpu.make_async_copy(
                    table_h.at[pl.ds(0, IDXW)],
                    rows.at[pl.ds(j * IDXW, IDXW)], sem).wait()

        def process(g, idxv, rows):
            for r in range(CHUNK):
                a0, a1, a2, a3 = row_accum(rows, r)
                cv = seg_count(idxv, r)
                inv = jnp.where(cv > 0, 1.0 / cv.astype(jnp.float32), 0.0)
                row = g * CHUNK + r
                pooled[row, pl.ds(0, L)] = a0 * inv
                pooled[row, pl.ds(L, L)] = a1 * inv
                pooled[row, pl.ds(2 * L, L)] = a2 * inv
                pooled[row, pl.ds(3 * L, L)] = a3 * inv

        stage(0, idxv0, rows0, sem0)

        def pair(i, carry):
            g = 2 * i
            stage(g + 1, idxv1, rows1, sem1)
            drain(rows0, sem0)
            process(g, idxv0, rows0)

            @pl.when(g + 2 < NCHUNK)
            def _():
                stage(g + 2, idxv0, rows0, sem0)

            drain(rows1, sem1)
            process(g + 1, idxv1, rows1)
            return carry

        lax.fori_loop(0, NCHUNK // 2, pair, 0)

        base = wid * NB
        pltpu.sync_copy(c0i_h.at[pl.ds(base, NB)], cidxv)
        pltpu.async_copy(cat0_h.at[cidxv], cr0, sem0).wait()
        pltpu.sync_copy(c1i_h.at[pl.ds(base, NB)], cidxv)
        pltpu.async_copy(cat1_h.at[cidxv], cr1, sem0).wait()
        pltpu.sync_copy(c2i_h.at[pl.ds(base, NB)], cidxv)
        pltpu.async_copy(cat2_h.at[cidxv], cr2, sem0).wait()

        def cbody(r, carry):
            for j in range(D // L):
                sl = pl.ds(L * j, L)
                pooled[r, sl] = pooled[r, sl] + cr0[r, sl] + cr1[r, sl] + cr2[r, sl]
            return carry

        lax.fori_loop(0, NB, cbody, 0)
        pltpu.sync_copy(pooled, out_h.at[pl.ds(base, NB)])

    return k(table2, idx1d, cidx0, cidx1, cidx2, cat0, cat1, cat2)


def _linear(x, W, b):
    BM = 512

    def mm(x_ref, w_ref, b_ref, o_ref):
        o_ref[...] = lax.dot_general(
            x_ref[...], w_ref[...], (((1,), (1,)), ((), ())),
            preferred_element_type=jnp.float32) + b_ref[...]

    return pl.pallas_call(
        mm,
        grid=(B // BM,),
        in_specs=[
            pl.BlockSpec((BM, D), lambda i: (i, 0)),
            pl.BlockSpec((NUM_CLASSES, D), lambda i: (0, 0)),
            pl.BlockSpec((1, NUM_CLASSES), lambda i: (0, 0)),
        ],
        out_specs=pl.BlockSpec((BM, NUM_CLASSES), lambda i: (i, 0)),
        out_shape=jax.ShapeDtypeStruct((B, NUM_CLASSES), jnp.float32),
    )(x, W, b.reshape(1, NUM_CLASSES))


_PERM = tuple(range(0, 2 * L, 2)) + tuple(range(1, 2 * L, 2)) + \
    tuple(range(2 * L, 4 * L, 2)) + tuple(range(2 * L + 1, 4 * L, 2))


def kernel(encoded_text, additional_inputs, emb_table, cat0, cat1, cat2, W, b):
    t128 = _format_table(emb_table.T)
    table2 = t128.reshape(4 * QUART, D // 2)
    idx1d = encoded_text.reshape(B * S)
    cidx0 = additional_inputs[:, 0]
    cidx1 = additional_inputs[:, 1]
    cidx2 = additional_inputs[:, 2]
    perm = jnp.array(_PERM, dtype=jnp.int32)
    pooled = _sc_pool(table2, idx1d, cidx0, cidx1, cidx2,
                      cat0[:, perm], cat1[:, perm], cat2[:, perm])
    return _linear(pooled, W[:, perm], b)


# speedup vs baseline: 2.0639x; 1.0086x over previous
import functools

import jax
import jax.numpy as jnp
from jax import lax
from jax.experimental import pallas as pl
from jax.experimental.pallas import tpu as pltpu
from jax.experimental.pallas import tpu_sc as plsc

VOCAB = 1000000
QUART = 1 << 18
B = 4096
S = 200
D = 64
NUM_CLASSES = 1000
L = 16
NC = 2
NS = 16
NW = NC * NS
NB = B // NW
CHUNK = 2
NCHUNK = NB // CHUNK
IDXW = 80
NSTREAM = CHUNK * S // IDXW
TOK = CHUNK * S


def _format_table(tableT):
    BN = 8192
    ngrid = QUART // BN
    last = pl.cdiv(VOCAB, BN) - 1

    def tr(a0_ref, a1_ref, a2_ref, a3_ref, o_ref):
        for q, a_ref in enumerate((a0_ref, a1_ref, a2_ref, a3_ref)):
            w = pltpu.bitcast(a_ref[...].astype(jnp.bfloat16), jnp.int32)
            o_ref[:, (D // 2) * q:(D // 2) * (q + 1)] = w.T

    def mk_map(q):
        return lambda i: (0, jnp.minimum(i + q * ngrid, last))

    return pl.pallas_call(
        tr,
        grid=(ngrid,),
        in_specs=[pl.BlockSpec((D, BN), mk_map(q)) for q in range(4)],
        out_specs=pl.BlockSpec((BN, 2 * D), lambda i: (i, 0)),
        out_shape=jax.ShapeDtypeStruct((QUART, 2 * D), jnp.int32),
    )(tableT, tableT, tableT, tableT)


def _sc_pool(table2, idx1d, cidx0, cidx1, cidx2, cat0, cat1, cat2):
    mesh = plsc.VectorSubcoreMesh(
        core_axis_name="c", subcore_axis_name="s",
        num_cores=NC, num_subcores=NS)

    @functools.partial(
        pl.kernel,
        out_type=jax.ShapeDtypeStruct((B, D), jnp.float32),
        mesh=mesh,
        compiler_params=pltpu.CompilerParams(
            needs_layout_passes=False, use_tc_tiling_on_sc=False),
        scratch_types=[
            pltpu.VMEM((TOK,), jnp.int32),
            pltpu.VMEM((TOK,), jnp.int32),
            pltpu.VMEM((TOK, D // 2), jnp.int32),
            pltpu.VMEM((TOK, D // 2), jnp.int32),
            pltpu.VMEM((NB,), jnp.int32),
            pltpu.VMEM((NB, D), jnp.float32),
            pltpu.VMEM((NB, D), jnp.float32),
            pltpu.VMEM((NB, D), jnp.float32),
            pltpu.VMEM((NB, D), jnp.float32),
            pltpu.SemaphoreType.DMA,
            pltpu.SemaphoreType.DMA,
        ],
    )
    def k(table_h, idx_h, c0i_h, c1i_h, c2i_h, cat0_h, cat1_h, cat2_h,
          out_h, idxv0, idxv1, rows0, rows1, cidxv, cr0, cr1, cr2, pooled,
          sem0, sem1):
        wid = lax.axis_index("s") * NC + lax.axis_index("c")
        lane = lax.iota(jnp.int32, L)

        def seg_count(idxv, r):
            cv = jnp.zeros((L,), jnp.int32)
            for t in range(S // L):
                v = idxv[pl.ds(r * S + L * t, L)]
                cv = cv + plsc.all_reduce_population_count(v != 0)
            tail = S % L
            v = idxv[pl.ds(r * S + S - L, L)]
            cv = cv + plsc.all_reduce_population_count((lane >= L - tail) & (v != 0))
            return cv

        def row_accum(rows, r):
            def tbody(t, a):
                a0, a1, a2, a3 = a
                base = r * S + t * 8
                for u in range(8):
                    tt = base + u
                    lo = plsc.bitcast(rows[tt, pl.ds(0, L)], jnp.bfloat16)
                    hi = plsc.bitcast(rows[tt, pl.ds(L, L)], jnp.bfloat16)
                    e0, o0 = plsc.unpack(lo, format=plsc.PackFormat.INTERLEAVED)
                    e1, o1 = plsc.unpack(hi, format=plsc.PackFormat.INTERLEAVED)
                    a0 = a0 + e0
                    a1 = a1 + o0
                    a2 = a2 + e1
                    a3 = a3 + o1
                return (a0, a1, a2, a3)
            z = jnp.zeros((L,), jnp.float32)
            return lax.fori_loop(0, S // 8, tbody, (z, z, z, z))

        def stage(g, idxv, rows, sem):
            ib = wid * (NB * S) + g * TOK
            pltpu.sync_copy(idx_h.at[pl.ds(ib, TOK)], idxv)
            for t in range(TOK // L):
                v = idxv[pl.ds(L * t, L)]
                idxv[pl.ds(L * t, L)] = \
                    ((v << 2) | (v >> 18)) & (4 * QUART - 1)
            for j in range(NSTREAM):
                pltpu.async_copy(
                    table_h.at[idxv.at[pl.ds(j * IDXW, IDXW)]],
                    rows.at[pl.ds(j * IDXW, IDXW)], sem)

        def drain(rows, sem):
            for j in range(NSTREAM):
                pltpu.make_async_copy(
                    table_h.at[pl.ds(0, IDXW)],
                    rows.at[pl.ds(j * IDXW, IDXW)], sem).wait()

        def process(g, idxv, rows):
            for r in range(CHUNK):
                a0, a1, a2, a3 = row_accum(rows, r)
                cv = seg_count(idxv, r)
                inv = jnp.where(cv > 0, 1.0 / cv.astype(jnp.float32), 0.0)
                row = g * CHUNK + r
                pooled[row, pl.ds(0, L)] = a0 * inv
                pooled[row, pl.ds(L, L)] = a1 * inv
                pooled[row, pl.ds(2 * L, L)] = a2 * inv
                pooled[row, pl.ds(3 * L, L)] = a3 * inv

        stage(0, idxv0, rows0, sem0)

        def pair(i, carry):
            g = 2 * i
            stage(g + 1, idxv1, rows1, sem1)
            drain(rows0, sem0)
            process(g, idxv0, rows0)

            @pl.when(g + 2 < NCHUNK)
            def _():
                stage(g + 2, idxv0, rows0, sem0)

            drain(rows1, sem1)
            process(g + 1, idxv1, rows1)
            return carry

        lax.fori_loop(0, NCHUNK // 2, pair, 0)

        base = wid * NB
        pltpu.sync_copy(c0i_h.at[pl.ds(base, NB)], cidxv)
        pltpu.async_copy(cat0_h.at[cidxv], cr0, sem0).wait()
        pltpu.sync_copy(c1i_h.at[pl.ds(base, NB)], cidxv)
        pltpu.async_copy(cat1_h.at[cidxv], cr1, sem0).wait()
        pltpu.sync_copy(c2i_h.at[pl.ds(base, NB)], cidxv)
        pltpu.async_copy(cat2_h.at[cidxv], cr2, sem0).wait()

        def cbody(r, carry):
            for j in range(D // L):
                sl = pl.ds(L * j, L)
                pooled[r, sl] = pooled[r, sl] + cr0[r, sl] + cr1[r, sl] + cr2[r, sl]
            return carry

        lax.fori_loop(0, NB, cbody, 0)
        pltpu.sync_copy(pooled, out_h.at[pl.ds(base, NB)])

    return k(table2, idx1d, cidx0, cidx1, cidx2, cat0, cat1, cat2)


def _linear(x, W, b):
    BM = 512

    def mm(x_ref, w_ref, b_ref, o_ref):
        o_ref[...] = lax.dot_general(
            x_ref[...], w_ref[...], (((1,), (1,)), ((), ())),
            preferred_element_type=jnp.float32) + b_ref[...]

    return pl.pallas_call(
        mm,
        grid=(B // BM,),
        in_specs=[
            pl.BlockSpec((BM, D), lambda i: (i, 0)),
            pl.BlockSpec((NUM_CLASSES, D), lambda i: (0, 0)),
            pl.BlockSpec((1, NUM_CLASSES), lambda i: (0, 0)),
        ],
        out_specs=pl.BlockSpec((BM, NUM_CLASSES), lambda i: (i, 0)),
        out_shape=jax.ShapeDtypeStruct((B, NUM_CLASSES), jnp.float32),
    )(x, W, b.reshape(1, NUM_CLASSES))


_PERM = tuple(range(0, 2 * L, 2)) + tuple(range(1, 2 * L, 2)) + \
    tuple(range(2 * L, 4 * L, 2)) + tuple(range(2 * L + 1, 4 * L, 2))


def kernel(encoded_text, additional_inputs, emb_table, cat0, cat1, cat2, W, b):
    t128 = _format_table(emb_table.T)
    table2 = t128.reshape(4 * QUART, D // 2)
    idx1d = encoded_text.reshape(B * S)
    cidx0 = additional_inputs[:, 0]
    cidx1 = additional_inputs[:, 1]
    cidx2 = additional_inputs[:, 2]
    perm = jnp.array(_PERM, dtype=jnp.int32)
    pooled = _sc_pool(table2, idx1d, cidx0, cidx1, cidx2,
                      cat0[:, perm], cat1[:, perm], cat2[:, perm])
    return _linear(pooled, W[:, perm], b)


# speedup vs baseline: 2.1409x; 1.0373x over previous
import functools

import jax
import jax.numpy as jnp
from jax import lax
from jax.experimental import pallas as pl
from jax.experimental.pallas import tpu as pltpu
from jax.experimental.pallas import tpu_sc as plsc

VOCAB = 1000000
QUART = 1 << 18
B = 4096
S = 200
D = 64
NUM_CLASSES = 1000
L = 16
NC = 2
NS = 16
NW = NC * NS
NB = B // NW
CHUNK = 4
NCHUNK = NB // CHUNK
IDXW = 80
NSTREAM = CHUNK * S // IDXW
TOK = CHUNK * S


def _format_table(tableT):
    BN = 8192
    ngrid = QUART // BN
    last = pl.cdiv(VOCAB, BN) - 1

    def tr(a0_ref, a1_ref, a2_ref, a3_ref, o_ref):
        for q, a_ref in enumerate((a0_ref, a1_ref, a2_ref, a3_ref)):
            w = pltpu.bitcast(a_ref[...].astype(jnp.bfloat16), jnp.int32)
            o_ref[:, (D // 2) * q:(D // 2) * (q + 1)] = w.T

    def mk_map(q):
        return lambda i: (0, jnp.minimum(i + q * ngrid, last))

    return pl.pallas_call(
        tr,
        grid=(ngrid,),
        in_specs=[pl.BlockSpec((D, BN), mk_map(q)) for q in range(4)],
        out_specs=pl.BlockSpec((BN, 2 * D), lambda i: (i, 0)),
        out_shape=jax.ShapeDtypeStruct((QUART, 2 * D), jnp.int32),
    )(tableT, tableT, tableT, tableT)


def _sc_pool(table2, idx1d, cidx0, cidx1, cidx2, cat0, cat1, cat2):
    mesh = plsc.VectorSubcoreMesh(
        core_axis_name="c", subcore_axis_name="s",
        num_cores=NC, num_subcores=NS)

    @functools.partial(
        pl.kernel,
        out_type=jax.ShapeDtypeStruct((B, D), jnp.float32),
        mesh=mesh,
        compiler_params=pltpu.CompilerParams(
            needs_layout_passes=False, use_tc_tiling_on_sc=False),
        scratch_types=[
            pltpu.VMEM((TOK,), jnp.int32),
            pltpu.VMEM((TOK,), jnp.int32),
            pltpu.VMEM((TOK, D // 2), jnp.int32),
            pltpu.VMEM((TOK, D // 2), jnp.int32),
            pltpu.VMEM((NB,), jnp.int32),
            pltpu.VMEM((NB, D), jnp.float32),
            pltpu.VMEM((NB, D), jnp.float32),
            pltpu.VMEM((NB, D), jnp.float32),
            pltpu.VMEM((NB, D), jnp.float32),
            pltpu.SemaphoreType.DMA,
            pltpu.SemaphoreType.DMA,
        ],
    )
    def k(table_h, idx_h, c0i_h, c1i_h, c2i_h, cat0_h, cat1_h, cat2_h,
          out_h, idxv0, idxv1, rows0, rows1, cidxv, cr0, cr1, cr2, pooled,
          sem0, sem1):
        wid = lax.axis_index("s") * NC + lax.axis_index("c")
        lane = lax.iota(jnp.int32, L)

        def seg_count(idxv, r):
            cv = jnp.zeros((L,), jnp.int32)
            for t in range(S // L):
                v = idxv[pl.ds(r * S + L * t, L)]
                cv = cv + plsc.all_reduce_population_count(v != 0)
            tail = S % L
            v = idxv[pl.ds(r * S + S - L, L)]
            cv = cv + plsc.all_reduce_population_count((lane >= L - tail) & (v != 0))
            return cv

        def row_accum(rows, r):
            def tbody(t, a):
                a0, a1, a2, a3 = a
                base = r * S + t * 8
                for u in range(8):
                    tt = base + u
                    lo = plsc.bitcast(rows[tt, pl.ds(0, L)], jnp.bfloat16)
                    hi = plsc.bitcast(rows[tt, pl.ds(L, L)], jnp.bfloat16)
                    e0, o0 = plsc.unpack(lo, format=plsc.PackFormat.INTERLEAVED)
                    e1, o1 = plsc.unpack(hi, format=plsc.PackFormat.INTERLEAVED)
                    a0 = a0 + e0
                    a1 = a1 + o0
                    a2 = a2 + e1
                    a3 = a3 + o1
                return (a0, a1, a2, a3)
            z = jnp.zeros((L,), jnp.float32)
            return lax.fori_loop(0, S // 8, tbody, (z, z, z, z))

        def stage(g, idxv, rows, sem):
            ib = wid * (NB * S) + g * TOK
            pltpu.sync_copy(idx_h.at[pl.ds(ib, TOK)], idxv)
            for t in range(TOK // L):
                v = idxv[pl.ds(L * t, L)]
                idxv[pl.ds(L * t, L)] = \
                    ((v << 2) | (v >> 18)) & (4 * QUART - 1)
            for j in range(NSTREAM):
                pltpu.async_copy(
                    table_h.at[idxv.at[pl.ds(j * IDXW, IDXW)]],
                    rows.at[pl.ds(j * IDXW, IDXW)], sem)

        def drain(rows, sem):
            for j in range(NSTREAM):
                pltpu.make_async_copy(
                    table_h.at[pl.ds(0, IDXW)],
                    rows.at[pl.ds(j * IDXW, IDXW)], sem).wait()

        def process(g, idxv, rows):
            for r in range(CHUNK):
                a0, a1, a2, a3 = row_accum(rows, r)
                cv = seg_count(idxv, r)
                inv = jnp.where(cv > 0, 1.0 / cv.astype(jnp.float32), 0.0)
                row = g * CHUNK + r
                pooled[row, pl.ds(0, L)] = a0 * inv
                pooled[row, pl.ds(L, L)] = a1 * inv
                pooled[row, pl.ds(2 * L, L)] = a2 * inv
                pooled[row, pl.ds(3 * L, L)] = a3 * inv

        stage(0, idxv0, rows0, sem0)

        def pair(i, carry):
            g = 2 * i
            stage(g + 1, idxv1, rows1, sem1)
            drain(rows0, sem0)
            process(g, idxv0, rows0)

            @pl.when(g + 2 < NCHUNK)
            def _():
                stage(g + 2, idxv0, rows0, sem0)

            drain(rows1, sem1)
            process(g + 1, idxv1, rows1)
            return carry

        lax.fori_loop(0, NCHUNK // 2, pair, 0)

        base = wid * NB
        pltpu.sync_copy(c0i_h.at[pl.ds(base, NB)], cidxv)
        pltpu.async_copy(cat0_h.at[cidxv], cr0, sem0).wait()
        pltpu.sync_copy(c1i_h.at[pl.ds(base, NB)], cidxv)
        pltpu.async_copy(cat1_h.at[cidxv], cr1, sem0).wait()
        pltpu.sync_copy(c2i_h.at[pl.ds(base, NB)], cidxv)
        pltpu.async_copy(cat2_h.at[cidxv], cr2, sem0).wait()

        def cbody(r, carry):
            for j in range(D // L):
                sl = pl.ds(L * j, L)
                pooled[r, sl] = pooled[r, sl] + cr0[r, sl] + cr1[r, sl] + cr2[r, sl]
            return carry

        lax.fori_loop(0, NB, cbody, 0)
        pltpu.sync_copy(pooled, out_h.at[pl.ds(base, NB)])

    return k(table2, idx1d, cidx0, cidx1, cidx2, cat0, cat1, cat2)


def _linear(x, W, b):
    BM = 512

    def mm(x_ref, w_ref, b_ref, o_ref):
        o_ref[...] = lax.dot_general(
            x_ref[...], w_ref[...], (((1,), (1,)), ((), ())),
            preferred_element_type=jnp.float32) + b_ref[...]

    return pl.pallas_call(
        mm,
        grid=(B // BM,),
        in_specs=[
            pl.BlockSpec((BM, D), lambda i: (i, 0)),
            pl.BlockSpec((NUM_CLASSES, D), lambda i: (0, 0)),
            pl.BlockSpec((1, NUM_CLASSES), lambda i: (0, 0)),
        ],
        out_specs=pl.BlockSpec((BM, NUM_CLASSES), lambda i: (i, 0)),
        out_shape=jax.ShapeDtypeStruct((B, NUM_CLASSES), jnp.float32),
    )(x, W, b.reshape(1, NUM_CLASSES))


_PERM = tuple(range(0, 2 * L, 2)) + tuple(range(1, 2 * L, 2)) + \
    tuple(range(2 * L, 4 * L, 2)) + tuple(range(2 * L + 1, 4 * L, 2))


def kernel(encoded_text, additional_inputs, emb_table, cat0, cat1, cat2, W, b):
    t128 = _format_table(emb_table.T)
    table2 = t128.reshape(4 * QUART, D // 2)
    idx1d = encoded_text.reshape(B * S)
    cidx0 = additional_inputs[:, 0]
    cidx1 = additional_inputs[:, 1]
    cidx2 = additional_inputs[:, 2]
    perm = jnp.array(_PERM, dtype=jnp.int32)
    pooled = _sc_pool(table2, idx1d, cidx0, cidx1, cidx2,
                      cat0[:, perm], cat1[:, perm], cat2[:, perm])
    return _linear(pooled, W[:, perm], b)
